# baseline (speedup 1.0000x reference)
import dataclasses
import functools

import jax
import jax.numpy as jnp
from jax import lax
from jax.experimental import pallas as pl
from jax.experimental.pallas import tpu as pltpu
from jax.experimental.pallas import tpu_sc as plsc

NS = 16
ACH = 128
BM = 1024


def _mesh():
    return plsc.VectorSubcoreMesh(core_axis_name="c", subcore_axis_name="s")


@functools.lru_cache(maxsize=None)
def _deg_call(N: int, E: int):
    e_per_tile = E // (2 * NS)

    @functools.partial(
        pl.kernel,
        out_type=jax.ShapeDtypeStruct((2 * NS, N), jnp.float32),
        mesh=_mesh(),
        scratch_types=[
            pltpu.VMEM((E // (2 * NS),), jnp.int32),
            pltpu.VMEM((N,), jnp.float32),
        ],
        compiler_params=dataclasses.replace(pltpu.CompilerParams(),
                                            needs_layout_passes=False),
    )
    def deg_kernel(dst_hbm, out_hbm, idx_d, hist_v):
        cid = lax.axis_index("c")
        sid = lax.axis_index("s")
        wid = cid * NS + sid
        pltpu.sync_copy(dst_hbm.at[pl.ds(wid * e_per_tile, e_per_tile)],
                        idx_d)

        @pl.loop(0, N, step=16)
        def _(i):
            hist_v[pl.ds(i, 16)] = jnp.zeros((16,), jnp.float32)

        ones_reg = jnp.full((16,), 1.0, jnp.float32)

        @pl.loop(0, e_per_tile, step=80)
        def _(k):
            for j in range(5):
                iv = idx_d[pl.ds(k + j * 16, 16)]
                plsc.addupdate_scatter(hist_v, [iv], ones_reg)

        pltpu.sync_copy(hist_v, out_hbm.at[wid])

    return deg_kernel


@functools.lru_cache(maxsize=None)
def _agg_call(N: int, EP: int, H: int):
    nch = EP // ACH // NS
    rpt = N // NS

    @functools.partial(
        pl.kernel,
        out_type=(jax.ShapeDtypeStruct((N, H), jnp.float32),
                  jax.ShapeDtypeStruct((N, H), jnp.float32)),
        mesh=_mesh(),
        scratch_types=[
            pltpu.VMEM((ACH,), jnp.int32),
            pltpu.VMEM((ACH,), jnp.int32),
            pltpu.VMEM((ACH,), jnp.int32),
            pltpu.VMEM((ACH,), jnp.int32),
            pltpu.VMEM((ACH, H), jnp.float32),
            pltpu.VMEM((ACH, H), jnp.float32),
            pltpu.SemaphoreType.DMA,
            pltpu.SemaphoreType.DMA,
            pltpu.VMEM_SHARED((N, H), jnp.float32),
        ],
    )
    def agg_kernel(ylo_hbm, yhi_hbm, src_hbm, dst_hbm, z_hbm,
                   outlo_hbm, outhi_hbm, is0, is1, id0, id1, rows0, rows1,
                   sem0, sem1, acc):
        cid = lax.axis_index("c")
        sid = lax.axis_index("s")
        pltpu.sync_copy(z_hbm, acc.at[pl.ds(sid * rpt, rpt)])
        plsc.subcore_barrier()
        cbase = sid * nch

        def process(y_hbm, out_hbm):
            pltpu.sync_copy(src_hbm.at[cbase], is0)
            pltpu.async_copy(y_hbm.at[is0], rows0, sem0)

            @pl.loop(0, nch, step=2)
            def _(j):
                pltpu.sync_copy(src_hbm.at[cbase + j + 1], is1)
                pltpu.async_copy(y_hbm.at[is1], rows1, sem1)
                pltpu.sync_copy(dst_hbm.at[cbase + j], id0)
                pltpu.make_async_copy(y_hbm.at[is0], rows0, sem0).wait()
                pltpu.sync_copy(rows0, acc.at[id0], add=True)

                @pl.when(j + 2 < nch)
                def _():
                    pltpu.sync_copy(src_hbm.at[cbase + j + 2], is0)
                    pltpu.async_copy(y_hbm.at[is0], rows0, sem0)

                pltpu.sync_copy(dst_hbm.at[cbase + j + 1], id1)
                pltpu.make_async_copy(y_hbm.at[is1], rows1, sem1).wait()
                pltpu.sync_copy(rows1, acc.at[id1], add=True)

            plsc.subcore_barrier()
            pltpu.sync_copy(acc.at[pl.ds(sid * rpt, rpt)],
                            out_hbm.at[pl.ds(sid * rpt, rpt)])

        @pl.when(cid == 0)
        def _():
            process(ylo_hbm, outlo_hbm)

        @pl.when(cid == 1)
        def _():
            process(yhi_hbm, outhi_hbm)

    return agg_kernel


def _mm1_body(x_ref, w_ref, o_ref):
    o_ref[...] = jnp.dot(x_ref[...], w_ref[...],
                         preferred_element_type=jnp.float32)


@functools.lru_cache(maxsize=None)
def _mm1(N, K, C):
    return pl.pallas_call(
        _mm1_body,
        grid=(N // BM,),
        in_specs=[pl.BlockSpec((BM, K), lambda i: (i, 0)),
                  pl.BlockSpec((K, C), lambda i: (0, 0))],
        out_specs=pl.BlockSpec((BM, C), lambda i: (i, 0)),
        out_shape=jax.ShapeDtypeStruct((N, C), jnp.float32),
    )


def _e1_body(dg_ref, xw_ref, dis_ref, ylo_ref, yhi_ref):
    ones32 = jnp.ones((dg_ref.shape[0], 1), jnp.float32)
    cnt = lax.dot_general(dg_ref[...], ones32, (((0,), (0,)), ((), ())),
                          preferred_element_type=jnp.float32)
    dis = lax.rsqrt(1.0 + cnt)
    dis_ref[...] = dis
    yw = dis * xw_ref[...]
    h = yw.shape[1] // 2
    ylo_ref[...] = yw[:, :h]
    yhi_ref[...] = yw[:, h:]


@functools.lru_cache(maxsize=None)
def _e1(N, C):
    H = C // 2
    return pl.pallas_call(
        _e1_body,
        grid=(N // BM,),
        in_specs=[pl.BlockSpec((2 * NS, BM), lambda i: (0, i)),
                  pl.BlockSpec((BM, C), lambda i: (i, 0))],
        out_specs=(pl.BlockSpec((BM, 1), lambda i: (i, 0)),
                   pl.BlockSpec((BM, H), lambda i: (i, 0)),
                   pl.BlockSpec((BM, H), lambda i: (i, 0))),
        out_shape=(jax.ShapeDtypeStruct((N, 1), jnp.float32),
                   jax.ShapeDtypeStruct((N, H), jnp.float32),
                   jax.ShapeDtypeStruct((N, H), jnp.float32)),
    )


def _k2_body(dis_ref, alo_ref, ahi_ref, xw_ref, b_ref, w_ref,
             xw2_ref, ylo_ref, yhi_ref):
    dis = dis_ref[...]
    agg = jnp.concatenate([alo_ref[...], ahi_ref[...]], axis=1)
    h = jnp.maximum(dis * agg + (dis * dis) * xw_ref[...] + b_ref[...], 0.0)
    xw2 = jnp.dot(h, w_ref[...], preferred_element_type=jnp.float32)
    xw2_ref[...] = xw2
    yw = dis * xw2
    hh = yw.shape[1] // 2
    ylo_ref[...] = yw[:, :hh]
    yhi_ref[...] = yw[:, hh:]


@functools.lru_cache(maxsize=None)
def _k2(N, C, C2):
    H = C // 2
    H2 = C2 // 2
    return pl.pallas_call(
        _k2_body,
        grid=(N // BM,),
        in_specs=[pl.BlockSpec((BM, 1), lambda i: (i, 0)),
                  pl.BlockSpec((BM, H), lambda i: (i, 0)),
                  pl.BlockSpec((BM, H), lambda i: (i, 0)),
                  pl.BlockSpec((BM, C), lambda i: (i, 0)),
                  pl.BlockSpec((1, C), lambda i: (0, 0)),
                  pl.BlockSpec((C, C2), lambda i: (0, 0))],
        out_specs=(pl.BlockSpec((BM, C2), lambda i: (i, 0)),
                   pl.BlockSpec((BM, H2), lambda i: (i, 0)),
                   pl.BlockSpec((BM, H2), lambda i: (i, 0))),
        out_shape=(jax.ShapeDtypeStruct((N, C2), jnp.float32),
                   jax.ShapeDtypeStruct((N, H2), jnp.float32),
                   jax.ShapeDtypeStruct((N, H2), jnp.float32)),
    )


def _k3_body(dis_ref, alo_ref, ahi_ref, xw_ref, b_ref, w_ref, blin_ref,
             o_ref):
    dis = dis_ref[...]
    agg = jnp.concatenate([alo_ref[...], ahi_ref[...]], axis=1)
    h = jnp.maximum(dis * agg + (dis * dis) * xw_ref[...] + b_ref[...], 0.0)
    o_ref[...] = jnp.dot(h, w_ref[...],
                         preferred_element_type=jnp.float32) + blin_ref[...]


@functools.lru_cache(maxsize=None)
def _k3(N, C, O):
    H = C // 2
    return pl.pallas_call(
        _k3_body,
        grid=(N // BM,),
        in_specs=[pl.BlockSpec((BM, 1), lambda i: (i, 0)),
                  pl.BlockSpec((BM, H), lambda i: (i, 0)),
                  pl.BlockSpec((BM, H), lambda i: (i, 0)),
                  pl.BlockSpec((BM, C), lambda i: (i, 0)),
                  pl.BlockSpec((1, C), lambda i: (0, 0)),
                  pl.BlockSpec((C, O), lambda i: (0, 0)),
                  pl.BlockSpec((1, O), lambda i: (0, 0))],
        out_specs=pl.BlockSpec((BM, O), lambda i: (i, 0)),
        out_shape=jax.ShapeDtypeStruct((N, O), jnp.float32),
    )


def kernel(x, edge_index, W1, b1, W2, b2, Wlin, blin):
    N, Cin = x.shape
    E = edge_index.shape[1]
    C = W1.shape[1]
    C2 = W2.shape[1]
    O = Wlin.shape[1]
    H = C // 2

    NP = ((N + BM - 1) // BM) * BM

    xp = jnp.pad(x, ((0, NP - N), (0, 0)))
    src = edge_index[0].astype(jnp.int32)
    dst = edge_index[1].astype(jnp.int32)
    z128 = jnp.zeros((NP // NS, H), jnp.float32)

    grp = NS * ACH * 8
    EP = ((E + grp - 1) // grp) * grp
    srcp = jnp.concatenate(
        [src, jnp.zeros((EP - E,), jnp.int32)]).reshape(-1, ACH)
    dstp = jnp.concatenate(
        [dst, jnp.full((EP - E,), NP - 1, jnp.int32)]).reshape(-1, ACH)

    degp = _deg_call(NP, E)(dst)
    xw1 = _mm1(NP, Cin, C)(xp, W1)
    dis, y1lo, y1hi = _e1(NP, C)(degp, xw1)
    a1lo, a1hi = _agg_call(NP, EP, H)(y1lo, y1hi, srcp, dstp, z128)
    xw2, y2lo, y2hi = _k2(NP, C, C2)(dis, a1lo, a1hi, xw1,
                                     b1.reshape(1, -1), W2)
    a2lo, a2hi = _agg_call(NP, EP, C2 // 2)(y2lo, y2hi, srcp, dstp, z128)
    out = _k3(NP, C2, O)(dis, a2lo, a2hi, xw2,
                         b2.reshape(1, -1), Wlin, blin.reshape(1, -1))
    return out[:N]


# speedup vs baseline: 1.0956x; 1.0956x over previous
import dataclasses
import functools

import jax
import jax.numpy as jnp
from jax import lax
from jax.experimental import pallas as pl
from jax.experimental.pallas import tpu as pltpu
from jax.experimental.pallas import tpu_sc as plsc

NS = 16
CH = 80
BM = 1024


def _mesh():
    return plsc.VectorSubcoreMesh(core_axis_name="c", subcore_axis_name="s")


def _no_layout():
    return dataclasses.replace(pltpu.CompilerParams(),
                               needs_layout_passes=False)


@functools.lru_cache(maxsize=None)
def _deg_call(N: int, E: int):
    e_per_tile = E // (2 * NS)

    @functools.partial(
        pl.kernel,
        out_type=jax.ShapeDtypeStruct((2 * NS, N), jnp.float32),
        mesh=_mesh(),
        scratch_types=[
            pltpu.VMEM((E // (2 * NS),), jnp.int32),
            pltpu.VMEM((N,), jnp.float32),
        ],
        compiler_params=_no_layout(),
    )
    def deg_kernel(dst_hbm, out_hbm, idx_d, hist_v):
        cid = lax.axis_index("c")
        sid = lax.axis_index("s")
        wid = cid * NS + sid
        pltpu.sync_copy(dst_hbm.at[pl.ds(wid * e_per_tile, e_per_tile)],
                        idx_d)

        @pl.loop(0, N, step=16)
        def _(i):
            hist_v[pl.ds(i, 16)] = jnp.zeros((16,), jnp.float32)

        ones_reg = jnp.full((16,), 1.0, jnp.float32)

        @pl.loop(0, e_per_tile, step=80)
        def _(k):
            for j in range(5):
                iv = idx_d[pl.ds(k + j * 16, 16)]
                plsc.addupdate_scatter(hist_v, [iv], ones_reg)

        pltpu.sync_copy(hist_v, out_hbm.at[wid])

    return deg_kernel


@functools.lru_cache(maxsize=None)
def _agg_call(N: int, E: int, H: int):
    e_per_tile = E // NS
    rpt = N // NS

    @functools.partial(
        pl.kernel,
        out_type=(jax.ShapeDtypeStruct((N, H), jnp.float32),
                  jax.ShapeDtypeStruct((N, H), jnp.float32)),
        mesh=_mesh(),
        scratch_types=[
            pltpu.VMEM((CH,), jnp.int32),
            pltpu.VMEM((CH,), jnp.int32),
            pltpu.VMEM((CH, H), jnp.float32),
            pltpu.VMEM_SHARED((N, H), jnp.float32),
            pltpu.SemaphoreType.DMA,
        ],
    )
    def agg_kernel(y2_hbm, s0_hbm, s1_hbm, dst_hbm, z_hbm,
                   outlo_hbm, outhi_hbm, idx_s, idx_d, rows, acc, sem):
        cid = lax.axis_index("c")
        sid = lax.axis_index("s")
        pltpu.sync_copy(z_hbm, acc.at[pl.ds(sid * rpt, rpt)])
        plsc.subcore_barrier()
        tbase = sid * e_per_tile

        def process(s_hbm, out_hbm):
            @pl.loop(0, e_per_tile, step=CH)
            def _(k):
                b = tbase + k
                pltpu.sync_copy(s_hbm.at[pl.ds(b, CH)], idx_s)
                pltpu.sync_copy(dst_hbm.at[pl.ds(b, CH)], idx_d)
                pltpu.async_copy(y2_hbm.at[idx_s], rows, sem).wait()
                pltpu.sync_copy(rows, acc.at[idx_d], add=True)

            plsc.subcore_barrier()
            pltpu.sync_copy(acc.at[pl.ds(sid * rpt, rpt)],
                            out_hbm.at[pl.ds(sid * rpt, rpt)])

        @pl.when(cid == 0)
        def _():
            process(s0_hbm, outlo_hbm)

        @pl.when(cid == 1)
        def _():
            process(s1_hbm, outhi_hbm)

    return agg_kernel


def _mm1_body(x_ref, w_ref, o_ref):
    o_ref[...] = jnp.dot(x_ref[...], w_ref[...],
                         preferred_element_type=jnp.float32)


@functools.lru_cache(maxsize=None)
def _mm1(N, K, C):
    return pl.pallas_call(
        _mm1_body,
        grid=(N // BM,),
        in_specs=[pl.BlockSpec((BM, K), lambda i: (i, 0)),
                  pl.BlockSpec((K, C), lambda i: (0, 0))],
        out_specs=pl.BlockSpec((BM, C), lambda i: (i, 0)),
        out_shape=jax.ShapeDtypeStruct((N, C), jnp.float32),
    )


def _e1_body(dg_ref, xw_ref, dis_ref, y_ref):
    ones32 = jnp.ones((dg_ref.shape[0], 1), jnp.float32)
    cnt = lax.dot_general(dg_ref[...], ones32, (((0,), (0,)), ((), ())),
                          preferred_element_type=jnp.float32)
    dis = lax.rsqrt(1.0 + cnt)
    dis_ref[...] = dis
    y_ref[...] = dis * xw_ref[...]


@functools.lru_cache(maxsize=None)
def _e1(N, C):
    return pl.pallas_call(
        _e1_body,
        grid=(N // BM,),
        in_specs=[pl.BlockSpec((2 * NS, BM), lambda i: (0, i)),
                  pl.BlockSpec((BM, C), lambda i: (i, 0))],
        out_specs=(pl.BlockSpec((BM, 1), lambda i: (i, 0)),
                   pl.BlockSpec((BM, C), lambda i: (i, 0))),
        out_shape=(jax.ShapeDtypeStruct((N, 1), jnp.float32),
                   jax.ShapeDtypeStruct((N, C), jnp.float32)),
    )


def _k2_body(dis_ref, alo_ref, ahi_ref, xw_ref, b_ref, w_ref,
             xw2_ref, y_ref):
    dis = dis_ref[...]
    agg = jnp.concatenate([alo_ref[...], ahi_ref[...]], axis=1)
    h = jnp.maximum(dis * agg + (dis * dis) * xw_ref[...] + b_ref[...], 0.0)
    xw2 = jnp.dot(h, w_ref[...], preferred_element_type=jnp.float32)
    xw2_ref[...] = xw2
    y_ref[...] = dis * xw2


@functools.lru_cache(maxsize=None)
def _k2(N, C, C2):
    H = C // 2
    return pl.pallas_call(
        _k2_body,
        grid=(N // BM,),
        in_specs=[pl.BlockSpec((BM, 1), lambda i: (i, 0)),
                  pl.BlockSpec((BM, H), lambda i: (i, 0)),
                  pl.BlockSpec((BM, H), lambda i: (i, 0)),
                  pl.BlockSpec((BM, C), lambda i: (i, 0)),
                  pl.BlockSpec((1, C), lambda i: (0, 0)),
                  pl.BlockSpec((C, C2), lambda i: (0, 0))],
        out_specs=(pl.BlockSpec((BM, C2), lambda i: (i, 0)),
                   pl.BlockSpec((BM, C2), lambda i: (i, 0))),
        out_shape=(jax.ShapeDtypeStruct((N, C2), jnp.float32),
                   jax.ShapeDtypeStruct((N, C2), jnp.float32)),
    )


def _k3_body(dis_ref, alo_ref, ahi_ref, xw_ref, b_ref, w_ref, blin_ref,
             o_ref):
    dis = dis_ref[...]
    agg = jnp.concatenate([alo_ref[...], ahi_ref[...]], axis=1)
    h = jnp.maximum(dis * agg + (dis * dis) * xw_ref[...] + b_ref[...], 0.0)
    o_ref[...] = jnp.dot(h, w_ref[...],
                         preferred_element_type=jnp.float32) + blin_ref[...]


@functools.lru_cache(maxsize=None)
def _k3(N, C, O):
    H = C // 2
    return pl.pallas_call(
        _k3_body,
        grid=(N // BM,),
        in_specs=[pl.BlockSpec((BM, 1), lambda i: (i, 0)),
                  pl.BlockSpec((BM, H), lambda i: (i, 0)),
                  pl.BlockSpec((BM, H), lambda i: (i, 0)),
                  pl.BlockSpec((BM, C), lambda i: (i, 0)),
                  pl.BlockSpec((1, C), lambda i: (0, 0)),
                  pl.BlockSpec((C, O), lambda i: (0, 0)),
                  pl.BlockSpec((1, O), lambda i: (0, 0))],
        out_specs=pl.BlockSpec((BM, O), lambda i: (i, 0)),
        out_shape=jax.ShapeDtypeStruct((N, O), jnp.float32),
    )


def kernel(x, edge_index, W1, b1, W2, b2, Wlin, blin):
    N, Cin = x.shape
    E = edge_index.shape[1]
    C = W1.shape[1]
    C2 = W2.shape[1]
    O = Wlin.shape[1]
    H = C // 2

    NP = ((N + BM - 1) // BM) * BM

    xp = jnp.pad(x, ((0, NP - N), (0, 0)))
    src = edge_index[0].astype(jnp.int32)
    dst = edge_index[1].astype(jnp.int32)
    s0 = src * 2
    s1 = src * 2 + 1
    z = jnp.zeros((NP // NS, H), jnp.float32)

    degp = _deg_call(NP, E)(dst)
    xw1 = _mm1(NP, Cin, C)(xp, W1)
    dis, y1 = _e1(NP, C)(degp, xw1)
    a1lo, a1hi = _agg_call(NP, E, H)(y1.reshape(2 * NP, H), s0, s1, dst, z)
    xw2, y2 = _k2(NP, C, C2)(dis, a1lo, a1hi, xw1, b1.reshape(1, -1), W2)
    a2lo, a2hi = _agg_call(NP, E, C2 // 2)(y2.reshape(2 * NP, C2 // 2),
                                           s0, s1, dst, z)
    out = _k3(NP, C2, O)(dis, a2lo, a2hi, xw2, b2.reshape(1, -1),
                         Wlin, blin.reshape(1, -1))
    return out[:N]


# speedup vs baseline: 1.4991x; 1.3683x over previous
import dataclasses
import functools

import jax
import jax.numpy as jnp
from jax import lax
from jax.experimental import pallas as pl
from jax.experimental.pallas import tpu as pltpu
from jax.experimental.pallas import tpu_sc as plsc

NS = 16
CH = 80
BM = 1024


def _mesh():
    return plsc.VectorSubcoreMesh(core_axis_name="c", subcore_axis_name="s")


def _no_layout():
    return dataclasses.replace(pltpu.CompilerParams(),
                               needs_layout_passes=False)


@functools.lru_cache(maxsize=None)
def _deg_call(N: int, E: int):
    e_per_tile = E // (2 * NS)

    @functools.partial(
        pl.kernel,
        out_type=jax.ShapeDtypeStruct((2 * NS, N), jnp.float32),
        mesh=_mesh(),
        scratch_types=[
            pltpu.VMEM((E // (2 * NS),), jnp.int32),
            pltpu.VMEM((N,), jnp.float32),
        ],
        compiler_params=_no_layout(),
    )
    def deg_kernel(dst_hbm, out_hbm, idx_d, hist_v):
        cid = lax.axis_index("c")
        sid = lax.axis_index("s")
        wid = cid * NS + sid
        pltpu.sync_copy(dst_hbm.at[pl.ds(wid * e_per_tile, e_per_tile)],
                        idx_d)

        @pl.loop(0, N, step=16)
        def _(i):
            hist_v[pl.ds(i, 16)] = jnp.zeros((16,), jnp.float32)

        ones_reg = jnp.full((16,), 1.0, jnp.float32)

        @pl.loop(0, e_per_tile, step=80)
        def _(k):
            for j in range(5):
                iv = idx_d[pl.ds(k + j * 16, 16)]
                plsc.addupdate_scatter(hist_v, [iv], ones_reg)

        pltpu.sync_copy(hist_v, out_hbm.at[wid])

    return deg_kernel


@functools.lru_cache(maxsize=None)
def _agg_call(N: int, E: int, H: int):
    e_per_tile = E // NS
    rpt = N // NS
    BCH = 10 * CH

    @functools.partial(
        pl.kernel,
        out_type=(jax.ShapeDtypeStruct((N, H), jnp.float32),
                  jax.ShapeDtypeStruct((N, H), jnp.float32)),
        mesh=_mesh(),
        scratch_types=[
            pltpu.VMEM((BCH,), jnp.int32),
            pltpu.VMEM((BCH,), jnp.int32),
            pltpu.VMEM((CH,), jnp.int32),
            pltpu.VMEM((CH,), jnp.int32),
            pltpu.VMEM((CH, H), jnp.float32),
            pltpu.VMEM_SHARED((N, H), jnp.float32),
            pltpu.SemaphoreType.DMA,
        ],
    )
    def agg_kernel(y2_hbm, s0_hbm, s1_hbm, dst_hbm, z_hbm,
                   outlo_hbm, outhi_hbm, sbatch, dbatch, idx_s, idx_d,
                   rows, acc, sem):
        cid = lax.axis_index("c")
        sid = lax.axis_index("s")
        pltpu.sync_copy(z_hbm, acc.at[pl.ds(sid * rpt, rpt)])
        plsc.subcore_barrier()
        tbase = sid * e_per_tile

        def process(s_hbm, out_hbm):
            @pl.loop(0, e_per_tile, step=BCH)
            def _(kb):
                pltpu.sync_copy(s_hbm.at[pl.ds(tbase + kb, BCH)], sbatch)
                pltpu.sync_copy(dst_hbm.at[pl.ds(tbase + kb, BCH)], dbatch)
                for j in range(BCH // CH):
                    for i in range(CH // 16):
                        o = j * CH + i * 16
                        idx_s[pl.ds(i * 16, 16)] = sbatch[pl.ds(o, 16)]
                        idx_d[pl.ds(i * 16, 16)] = dbatch[pl.ds(o, 16)]
                    pltpu.async_copy(y2_hbm.at[idx_s], rows, sem).wait()
                    pltpu.sync_copy(rows, acc.at[idx_d], add=True)

            plsc.subcore_barrier()
            pltpu.sync_copy(acc.at[pl.ds(sid * rpt, rpt)],
                            out_hbm.at[pl.ds(sid * rpt, rpt)])

        @pl.when(cid == 0)
        def _():
            process(s0_hbm, outlo_hbm)

        @pl.when(cid == 1)
        def _():
            process(s1_hbm, outhi_hbm)

    return agg_kernel


def _mm1_body(x_ref, w_ref, o_ref):
    o_ref[...] = jnp.dot(x_ref[...], w_ref[...],
                         preferred_element_type=jnp.float32)


@functools.lru_cache(maxsize=None)
def _mm1(N, K, C):
    return pl.pallas_call(
        _mm1_body,
        grid=(N // BM,),
        in_specs=[pl.BlockSpec((BM, K), lambda i: (i, 0)),
                  pl.BlockSpec((K, C), lambda i: (0, 0))],
        out_specs=pl.BlockSpec((BM, C), lambda i: (i, 0)),
        out_shape=jax.ShapeDtypeStruct((N, C), jnp.float32),
    )


def _e1_body(dg_ref, xw_ref, dis_ref, y_ref):
    ones32 = jnp.ones((dg_ref.shape[0], 1), jnp.float32)
    cnt = lax.dot_general(dg_ref[...], ones32, (((0,), (0,)), ((), ())),
                          preferred_element_type=jnp.float32)
    dis = lax.rsqrt(1.0 + cnt)
    dis_ref[...] = dis
    y_ref[...] = dis * xw_ref[...]


@functools.lru_cache(maxsize=None)
def _e1(N, C):
    return pl.pallas_call(
        _e1_body,
        grid=(N // BM,),
        in_specs=[pl.BlockSpec((2 * NS, BM), lambda i: (0, i)),
                  pl.BlockSpec((BM, C), lambda i: (i, 0))],
        out_specs=(pl.BlockSpec((BM, 1), lambda i: (i, 0)),
                   pl.BlockSpec((BM, C), lambda i: (i, 0))),
        out_shape=(jax.ShapeDtypeStruct((N, 1), jnp.float32),
                   jax.ShapeDtypeStruct((N, C), jnp.float32)),
    )


def _k2_body(dis_ref, alo_ref, ahi_ref, xw_ref, b_ref, w_ref,
             xw2_ref, y_ref):
    dis = dis_ref[...]
    agg = jnp.concatenate([alo_ref[...], ahi_ref[...]], axis=1)
    h = jnp.maximum(dis * agg + (dis * dis) * xw_ref[...] + b_ref[...], 0.0)
    xw2 = jnp.dot(h, w_ref[...], preferred_element_type=jnp.float32)
    xw2_ref[...] = xw2
    y_ref[...] = dis * xw2


@functools.lru_cache(maxsize=None)
def _k2(N, C, C2):
    H = C // 2
    return pl.pallas_call(
        _k2_body,
        grid=(N // BM,),
        in_specs=[pl.BlockSpec((BM, 1), lambda i: (i, 0)),
                  pl.BlockSpec((BM, H), lambda i: (i, 0)),
                  pl.BlockSpec((BM, H), lambda i: (i, 0)),
                  pl.BlockSpec((BM, C), lambda i: (i, 0)),
                  pl.BlockSpec((1, C), lambda i: (0, 0)),
                  pl.BlockSpec((C, C2), lambda i: (0, 0))],
        out_specs=(pl.BlockSpec((BM, C2), lambda i: (i, 0)),
                   pl.BlockSpec((BM, C2), lambda i: (i, 0))),
        out_shape=(jax.ShapeDtypeStruct((N, C2), jnp.float32),
                   jax.ShapeDtypeStruct((N, C2), jnp.float32)),
    )


def _k3_body(dis_ref, alo_ref, ahi_ref, xw_ref, b_ref, w_ref, blin_ref,
             o_ref):
    dis = dis_ref[...]
    agg = jnp.concatenate([alo_ref[...], ahi_ref[...]], axis=1)
    h = jnp.maximum(dis * agg + (dis * dis) * xw_ref[...] + b_ref[...], 0.0)
    o_ref[...] = jnp.dot(h, w_ref[...],
                         preferred_element_type=jnp.float32) + blin_ref[...]


@functools.lru_cache(maxsize=None)
def _k3(N, C, O):
    H = C // 2
    return pl.pallas_call(
        _k3_body,
        grid=(N // BM,),
        in_specs=[pl.BlockSpec((BM, 1), lambda i: (i, 0)),
                  pl.BlockSpec((BM, H), lambda i: (i, 0)),
                  pl.BlockSpec((BM, H), lambda i: (i, 0)),
                  pl.BlockSpec((BM, C), lambda i: (i, 0)),
                  pl.BlockSpec((1, C), lambda i: (0, 0)),
                  pl.BlockSpec((C, O), lambda i: (0, 0)),
                  pl.BlockSpec((1, O), lambda i: (0, 0))],
        out_specs=pl.BlockSpec((BM, O), lambda i: (i, 0)),
        out_shape=jax.ShapeDtypeStruct((N, O), jnp.float32),
    )


def kernel(x, edge_index, W1, b1, W2, b2, Wlin, blin):
    N, Cin = x.shape
    E = edge_index.shape[1]
    C = W1.shape[1]
    C2 = W2.shape[1]
    O = Wlin.shape[1]
    H = C // 2

    NP = ((N + BM - 1) // BM) * BM

    xp = jnp.pad(x, ((0, NP - N), (0, 0)))
    src = edge_index[0].astype(jnp.int32)
    dst = edge_index[1].astype(jnp.int32)
    s0 = src * 2
    s1 = src * 2 + 1
    z = jnp.zeros((NP // NS, H), jnp.float32)

    degp = _deg_call(NP, E)(dst)
    xw1 = _mm1(NP, Cin, C)(xp, W1)
    dis, y1 = _e1(NP, C)(degp, xw1)
    a1lo, a1hi = _agg_call(NP, E, H)(y1.reshape(2 * NP, H), s0, s1, dst, z)
    xw2, y2 = _k2(NP, C, C2)(dis, a1lo, a1hi, xw1, b1.reshape(1, -1), W2)
    a2lo, a2hi = _agg_call(NP, E, C2 // 2)(y2.reshape(2 * NP, C2 // 2),
                                           s0, s1, dst, z)
    out = _k3(NP, C2, O)(dis, a2lo, a2hi, xw2, b2.reshape(1, -1),
                         Wlin, blin.reshape(1, -1))
    return out[:N]


# speedup vs baseline: 2.2037x; 1.4700x over previous
import dataclasses
import functools

import jax
import jax.numpy as jnp
from jax import lax
from jax.experimental import pallas as pl
from jax.experimental.pallas import tpu as pltpu
from jax.experimental.pallas import tpu_sc as plsc

NS = 16
CH = 80
BM = 1024


def _mesh():
    return plsc.VectorSubcoreMesh(core_axis_name="c", subcore_axis_name="s")


def _no_layout():
    return dataclasses.replace(pltpu.CompilerParams(),
                               needs_layout_passes=False)


@functools.lru_cache(maxsize=None)
def _deg_call(N: int, E: int):
    e_per_tile = E // (2 * NS)

    @functools.partial(
        pl.kernel,
        out_type=jax.ShapeDtypeStruct((2 * NS, N), jnp.float32),
        mesh=_mesh(),
        scratch_types=[
            pltpu.VMEM((E // (2 * NS),), jnp.int32),
            pltpu.VMEM((N,), jnp.float32),
        ],
        compiler_params=_no_layout(),
    )
    def deg_kernel(dst_hbm, out_hbm, idx_d, hist_v):
        cid = lax.axis_index("c")
        sid = lax.axis_index("s")
        wid = cid * NS + sid
        pltpu.sync_copy(dst_hbm.at[pl.ds(wid * e_per_tile, e_per_tile)],
                        idx_d)

        @pl.loop(0, N, step=16)
        def _(i):
            hist_v[pl.ds(i, 16)] = jnp.zeros((16,), jnp.float32)

        ones_reg = jnp.full((16,), 1.0, jnp.float32)

        @pl.loop(0, e_per_tile, step=80)
        def _(k):
            for j in range(5):
                iv = idx_d[pl.ds(k + j * 16, 16)]
                plsc.addupdate_scatter(hist_v, [iv], ones_reg)

        pltpu.sync_copy(hist_v, out_hbm.at[wid])

    return deg_kernel


@functools.lru_cache(maxsize=None)
def _agg_call(N: int, E: int, H: int):
    e_per_tile = E // NS
    rpt = N // NS
    BCH = 10 * CH

    @functools.partial(
        pl.kernel,
        out_type=(jax.ShapeDtypeStruct((N, H), jnp.float32),
                  jax.ShapeDtypeStruct((N, H), jnp.float32)),
        mesh=_mesh(),
        scratch_types=[
            pltpu.VMEM((BCH,), jnp.int32),
            pltpu.VMEM((BCH,), jnp.int32),
            pltpu.VMEM((CH,), jnp.int32),
            pltpu.VMEM((CH,), jnp.int32),
            pltpu.VMEM((CH,), jnp.int32),
            pltpu.VMEM((CH, H), jnp.float32),
            pltpu.VMEM((CH, H), jnp.float32),
            pltpu.VMEM_SHARED((N, H), jnp.float32),
            pltpu.SemaphoreType.DMA,
            pltpu.SemaphoreType.DMA,
        ],
    )
    def agg_kernel(y2_hbm, s0_hbm, s1_hbm, dst_hbm, z_hbm,
                   outlo_hbm, outhi_hbm, sbatch, dbatch, is0, is1, idx_d,
                   rows0, rows1, acc, sem0, sem1):
        cid = lax.axis_index("c")
        sid = lax.axis_index("s")
        pltpu.sync_copy(z_hbm, acc.at[pl.ds(sid * rpt, rpt)])
        plsc.subcore_barrier()
        tbase = sid * e_per_tile
        nj = BCH // CH
        isb = (is0, is1)
        rowsb = (rows0, rows1)
        semb = (sem0, sem1)

        def repack(dst_v, src_v, j):
            for i in range(CH // 16):
                dst_v[pl.ds(i * 16, 16)] = src_v[pl.ds(j * CH + i * 16, 16)]

        def process(s_hbm, out_hbm):
            @pl.loop(0, e_per_tile, step=BCH)
            def _(kb):
                pltpu.sync_copy(s_hbm.at[pl.ds(tbase + kb, BCH)], sbatch)
                pltpu.sync_copy(dst_hbm.at[pl.ds(tbase + kb, BCH)], dbatch)
                repack(is0, sbatch, 0)
                pltpu.async_copy(y2_hbm.at[is0], rows0, sem0)
                for j in range(nj):
                    cur, nxt = j % 2, (j + 1) % 2
                    if j + 1 < nj:
                        repack(isb[nxt], sbatch, j + 1)
                        pltpu.async_copy(y2_hbm.at[isb[nxt]], rowsb[nxt],
                                         semb[nxt])
                    pltpu.make_async_copy(y2_hbm.at[isb[cur]], rowsb[cur],
                                          semb[cur]).wait()
                    repack(idx_d, dbatch, j)
                    pltpu.sync_copy(rowsb[cur], acc.at[idx_d], add=True)

            plsc.subcore_barrier()
            pltpu.sync_copy(acc.at[pl.ds(sid * rpt, rpt)],
                            out_hbm.at[pl.ds(sid * rpt, rpt)])

        @pl.when(cid == 0)
        def _():
            process(s0_hbm, outlo_hbm)

        @pl.when(cid == 1)
        def _():
            process(s1_hbm, outhi_hbm)

    return agg_kernel


def _mm1_body(x_ref, w_ref, o_ref):
    o_ref[...] = jnp.dot(x_ref[...], w_ref[...],
                         preferred_element_type=jnp.float32)


@functools.lru_cache(maxsize=None)
def _mm1(N, K, C):
    return pl.pallas_call(
        _mm1_body,
        grid=(N // BM,),
        in_specs=[pl.BlockSpec((BM, K), lambda i: (i, 0)),
                  pl.BlockSpec((K, C), lambda i: (0, 0))],
        out_specs=pl.BlockSpec((BM, C), lambda i: (i, 0)),
        out_shape=jax.ShapeDtypeStruct((N, C), jnp.float32),
    )


def _e1_body(dg_ref, xw_ref, dis_ref, y_ref):
    ones32 = jnp.ones((dg_ref.shape[0], 1), jnp.float32)
    cnt = lax.dot_general(dg_ref[...], ones32, (((0,), (0,)), ((), ())),
                          preferred_element_type=jnp.float32)
    dis = lax.rsqrt(1.0 + cnt)
    dis_ref[...] = dis
    y_ref[...] = dis * xw_ref[...]


@functools.lru_cache(maxsize=None)
def _e1(N, C):
    return pl.pallas_call(
        _e1_body,
        grid=(N // BM,),
        in_specs=[pl.BlockSpec((2 * NS, BM), lambda i: (0, i)),
                  pl.BlockSpec((BM, C), lambda i: (i, 0))],
        out_specs=(pl.BlockSpec((BM, 1), lambda i: (i, 0)),
                   pl.BlockSpec((BM, C), lambda i: (i, 0))),
        out_shape=(jax.ShapeDtypeStruct((N, 1), jnp.float32),
                   jax.ShapeDtypeStruct((N, C), jnp.float32)),
    )


def _k2_body(dis_ref, alo_ref, ahi_ref, xw_ref, b_ref, w_ref,
             xw2_ref, y_ref):
    dis = dis_ref[...]
    agg = jnp.concatenate([alo_ref[...], ahi_ref[...]], axis=1)
    h = jnp.maximum(dis * agg + (dis * dis) * xw_ref[...] + b_ref[...], 0.0)
    xw2 = jnp.dot(h, w_ref[...], preferred_element_type=jnp.float32)
    xw2_ref[...] = xw2
    y_ref[...] = dis * xw2


@functools.lru_cache(maxsize=None)
def _k2(N, C, C2):
    H = C // 2
    return pl.pallas_call(
        _k2_body,
        grid=(N // BM,),
        in_specs=[pl.BlockSpec((BM, 1), lambda i: (i, 0)),
                  pl.BlockSpec((BM, H), lambda i: (i, 0)),
                  pl.BlockSpec((BM, H), lambda i: (i, 0)),
                  pl.BlockSpec((BM, C), lambda i: (i, 0)),
                  pl.BlockSpec((1, C), lambda i: (0, 0)),
                  pl.BlockSpec((C, C2), lambda i: (0, 0))],
        out_specs=(pl.BlockSpec((BM, C2), lambda i: (i, 0)),
                   pl.BlockSpec((BM, C2), lambda i: (i, 0))),
        out_shape=(jax.ShapeDtypeStruct((N, C2), jnp.float32),
                   jax.ShapeDtypeStruct((N, C2), jnp.float32)),
    )


def _k3_body(dis_ref, alo_ref, ahi_ref, xw_ref, b_ref, w_ref, blin_ref,
             o_ref):
    dis = dis_ref[...]
    agg = jnp.concatenate([alo_ref[...], ahi_ref[...]], axis=1)
    h = jnp.maximum(dis * agg + (dis * dis) * xw_ref[...] + b_ref[...], 0.0)
    o_ref[...] = jnp.dot(h, w_ref[...],
                         preferred_element_type=jnp.float32) + blin_ref[...]


@functools.lru_cache(maxsize=None)
def _k3(N, C, O):
    H = C // 2
    return pl.pallas_call(
        _k3_body,
        grid=(N // BM,),
        in_specs=[pl.BlockSpec((BM, 1), lambda i: (i, 0)),
                  pl.BlockSpec((BM, H), lambda i: (i, 0)),
                  pl.BlockSpec((BM, H), lambda i: (i, 0)),
                  pl.BlockSpec((BM, C), lambda i: (i, 0)),
                  pl.BlockSpec((1, C), lambda i: (0, 0)),
                  pl.BlockSpec((C, O), lambda i: (0, 0)),
                  pl.BlockSpec((1, O), lambda i: (0, 0))],
        out_specs=pl.BlockSpec((BM, O), lambda i: (i, 0)),
        out_shape=jax.ShapeDtypeStruct((N, O), jnp.float32),
    )


def kernel(x, edge_index, W1, b1, W2, b2, Wlin, blin):
    N, Cin = x.shape
    E = edge_index.shape[1]
    C = W1.shape[1]
    C2 = W2.shape[1]
    O = Wlin.shape[1]
    H = C // 2

    NP = ((N + BM - 1) // BM) * BM

    xp = jnp.pad(x, ((0, NP - N), (0, 0)))
    src = edge_index[0].astype(jnp.int32)
    dst = edge_index[1].astype(jnp.int32)
    s0 = src * 2
    s1 = src * 2 + 1
    z = jnp.zeros((NP // NS, H), jnp.float32)

    degp = _deg_call(NP, E)(dst)
    xw1 = _mm1(NP, Cin, C)(xp, W1)
    dis, y1 = _e1(NP, C)(degp, xw1)
    a1lo, a1hi = _agg_call(NP, E, H)(y1.reshape(2 * NP, H), s0, s1, dst, z)
    xw2, y2 = _k2(NP, C, C2)(dis, a1lo, a1hi, xw1, b1.reshape(1, -1), W2)
    a2lo, a2hi = _agg_call(NP, E, C2 // 2)(y2.reshape(2 * NP, C2 // 2),
                                           s0, s1, dst, z)
    out = _k3(NP, C2, O)(dis, a2lo, a2hi, xw2, b2.reshape(1, -1),
                         Wlin, blin.reshape(1, -1))
    return out[:N]


# speedup vs baseline: 2.4610x; 1.1168x over previous
import dataclasses
import functools

import jax
import jax.numpy as jnp
from jax import lax
from jax.experimental import pallas as pl
from jax.experimental.pallas import tpu as pltpu
from jax.experimental.pallas import tpu_sc as plsc

NS = 16
CH = 80
BM = 1024


def _mesh():
    return plsc.VectorSubcoreMesh(core_axis_name="c", subcore_axis_name="s")


def _no_layout():
    return dataclasses.replace(pltpu.CompilerParams(),
                               needs_layout_passes=False)


@functools.lru_cache(maxsize=None)
def _deg_call(N: int, E: int):
    e_per_tile = E // (2 * NS)

    @functools.partial(
        pl.kernel,
        out_type=jax.ShapeDtypeStruct((2 * NS, N), jnp.float32),
        mesh=_mesh(),
        scratch_types=[
            pltpu.VMEM((E // (2 * NS),), jnp.int32),
            pltpu.VMEM((N,), jnp.float32),
        ],
        compiler_params=_no_layout(),
    )
    def deg_kernel(dst_hbm, out_hbm, idx_d, hist_v):
        cid = lax.axis_index("c")
        sid = lax.axis_index("s")
        wid = cid * NS + sid
        pltpu.sync_copy(dst_hbm.at[pl.ds(wid * e_per_tile, e_per_tile)],
                        idx_d)

        @pl.loop(0, N, step=16)
        def _(i):
            hist_v[pl.ds(i, 16)] = jnp.zeros((16,), jnp.float32)

        ones_reg = jnp.full((16,), 1.0, jnp.float32)

        @pl.loop(0, e_per_tile, step=80)
        def _(k):
            for j in range(5):
                iv = idx_d[pl.ds(k + j * 16, 16)]
                plsc.addupdate_scatter(hist_v, [iv], ones_reg)

        pltpu.sync_copy(hist_v, out_hbm.at[wid])

    return deg_kernel


@functools.lru_cache(maxsize=None)
def _agg_call(N: int, E: int, H: int):
    e_per_tile = E // NS
    rpt = N // NS
    BCH = 50 * CH

    @functools.partial(
        pl.kernel,
        out_type=(jax.ShapeDtypeStruct((N, H), jnp.float32),
                  jax.ShapeDtypeStruct((N, H), jnp.float32)),
        mesh=_mesh(),
        scratch_types=[
            pltpu.VMEM((BCH,), jnp.int32),
            pltpu.VMEM((BCH,), jnp.int32),
            pltpu.VMEM((CH,), jnp.int32),
            pltpu.VMEM((CH,), jnp.int32),
            pltpu.VMEM((CH,), jnp.int32),
            pltpu.VMEM((CH, H), jnp.float32),
            pltpu.VMEM((CH, H), jnp.float32),
            pltpu.VMEM_SHARED((N, H), jnp.float32),
            pltpu.SemaphoreType.DMA,
            pltpu.SemaphoreType.DMA,
        ],
    )
    def agg_kernel(y2_hbm, s0_hbm, s1_hbm, dst_hbm, z_hbm,
                   outlo_hbm, outhi_hbm, sbatch, dbatch, is0, is1, idx_d,
                   rows0, rows1, acc, sem0, sem1):
        cid = lax.axis_index("c")
        sid = lax.axis_index("s")
        pltpu.sync_copy(z_hbm, acc.at[pl.ds(sid * rpt, rpt)])
        plsc.subcore_barrier()
        tbase = sid * e_per_tile
        nj = BCH // CH
        isb = (is0, is1)
        rowsb = (rows0, rows1)
        semb = (sem0, sem1)

        def repack(dst_v, src_v, j):
            for i in range(CH // 16):
                dst_v[pl.ds(i * 16, 16)] = src_v[pl.ds(j * CH + i * 16, 16)]

        def process(s_hbm, out_hbm):
            @pl.loop(0, e_per_tile, step=BCH)
            def _(kb):
                pltpu.sync_copy(s_hbm.at[pl.ds(tbase + kb, BCH)], sbatch)
                pltpu.sync_copy(dst_hbm.at[pl.ds(tbase + kb, BCH)], dbatch)
                repack(is0, sbatch, 0)
                pltpu.async_copy(y2_hbm.at[is0], rows0, sem0)
                for j in range(nj):
                    cur, nxt = j % 2, (j + 1) % 2
                    if j + 1 < nj:
                        repack(isb[nxt], sbatch, j + 1)
                        pltpu.async_copy(y2_hbm.at[isb[nxt]], rowsb[nxt],
                                         semb[nxt])
                    pltpu.make_async_copy(y2_hbm.at[isb[cur]], rowsb[cur],
                                          semb[cur]).wait()
                    repack(idx_d, dbatch, j)
                    pltpu.sync_copy(rowsb[cur], acc.at[idx_d], add=True)

            plsc.subcore_barrier()
            pltpu.sync_copy(acc.at[pl.ds(sid * rpt, rpt)],
                            out_hbm.at[pl.ds(sid * rpt, rpt)])

        @pl.when(cid == 0)
        def _():
            process(s0_hbm, outlo_hbm)

        @pl.when(cid == 1)
        def _():
            process(s1_hbm, outhi_hbm)

    return agg_kernel


def _mm1_body(x_ref, w_ref, o_ref):
    o_ref[...] = jnp.dot(x_ref[...], w_ref[...],
                         preferred_element_type=jnp.float32)


@functools.lru_cache(maxsize=None)
def _mm1(N, K, C):
    return pl.pallas_call(
        _mm1_body,
        grid=(N // BM,),
        in_specs=[pl.BlockSpec((BM, K), lambda i: (i, 0)),
                  pl.BlockSpec((K, C), lambda i: (0, 0))],
        out_specs=pl.BlockSpec((BM, C), lambda i: (i, 0)),
        out_shape=jax.ShapeDtypeStruct((N, C), jnp.float32),
    )


def _e1_body(dg_ref, xw_ref, dis_ref, y_ref):
    ones32 = jnp.ones((dg_ref.shape[0], 1), jnp.float32)
    cnt = lax.dot_general(dg_ref[...], ones32, (((0,), (0,)), ((), ())),
                          preferred_element_type=jnp.float32)
    dis = lax.rsqrt(1.0 + cnt)
    dis_ref[...] = dis
    y_ref[...] = dis * xw_ref[...]


@functools.lru_cache(maxsize=None)
def _e1(N, C):
    return pl.pallas_call(
        _e1_body,
        grid=(N // BM,),
        in_specs=[pl.BlockSpec((2 * NS, BM), lambda i: (0, i)),
                  pl.BlockSpec((BM, C), lambda i: (i, 0))],
        out_specs=(pl.BlockSpec((BM, 1), lambda i: (i, 0)),
                   pl.BlockSpec((BM, C), lambda i: (i, 0))),
        out_shape=(jax.ShapeDtypeStruct((N, 1), jnp.float32),
                   jax.ShapeDtypeStruct((N, C), jnp.float32)),
    )


def _k2_body(dis_ref, alo_ref, ahi_ref, xw_ref, b_ref, w_ref,
             xw2_ref, y_ref):
    dis = dis_ref[...]
    agg = jnp.concatenate([alo_ref[...], ahi_ref[...]], axis=1)
    h = jnp.maximum(dis * agg + (dis * dis) * xw_ref[...] + b_ref[...], 0.0)
    xw2 = jnp.dot(h, w_ref[...], preferred_element_type=jnp.float32)
    xw2_ref[...] = xw2
    y_ref[...] = dis * xw2


@functools.lru_cache(maxsize=None)
def _k2(N, C, C2):
    H = C // 2
    return pl.pallas_call(
        _k2_body,
        grid=(N // BM,),
        in_specs=[pl.BlockSpec((BM, 1), lambda i: (i, 0)),
                  pl.BlockSpec((BM, H), lambda i: (i, 0)),
                  pl.BlockSpec((BM, H), lambda i: (i, 0)),
                  pl.BlockSpec((BM, C), lambda i: (i, 0)),
                  pl.BlockSpec((1, C), lambda i: (0, 0)),
                  pl.BlockSpec((C, C2), lambda i: (0, 0))],
        out_specs=(pl.BlockSpec((BM, C2), lambda i: (i, 0)),
                   pl.BlockSpec((BM, C2), lambda i: (i, 0))),
        out_shape=(jax.ShapeDtypeStruct((N, C2), jnp.float32),
                   jax.ShapeDtypeStruct((N, C2), jnp.float32)),
    )


def _k3_body(dis_ref, alo_ref, ahi_ref, xw_ref, b_ref, w_ref, blin_ref,
             o_ref):
    dis = dis_ref[...]
    agg = jnp.concatenate([alo_ref[...], ahi_ref[...]], axis=1)
    h = jnp.maximum(dis * agg + (dis * dis) * xw_ref[...] + b_ref[...], 0.0)
    o_ref[...] = jnp.dot(h, w_ref[...],
                         preferred_element_type=jnp.float32) + blin_ref[...]


@functools.lru_cache(maxsize=None)
def _k3(N, C, O):
    H = C // 2
    return pl.pallas_call(
        _k3_body,
        grid=(N // BM,),
        in_specs=[pl.BlockSpec((BM, 1), lambda i: (i, 0)),
                  pl.BlockSpec((BM, H), lambda i: (i, 0)),
                  pl.BlockSpec((BM, H), lambda i: (i, 0)),
                  pl.BlockSpec((BM, C), lambda i: (i, 0)),
                  pl.BlockSpec((1, C), lambda i: (0, 0)),
                  pl.BlockSpec((C, O), lambda i: (0, 0)),
                  pl.BlockSpec((1, O), lambda i: (0, 0))],
        out_specs=pl.BlockSpec((BM, O), lambda i: (i, 0)),
        out_shape=jax.ShapeDtypeStruct((N, O), jnp.float32),
    )


def kernel(x, edge_index, W1, b1, W2, b2, Wlin, blin):
    N, Cin = x.shape
    E = edge_index.shape[1]
    C = W1.shape[1]
    C2 = W2.shape[1]
    O = Wlin.shape[1]
    H = C // 2

    NP = ((N + BM - 1) // BM) * BM

    xp = jnp.pad(x, ((0, NP - N), (0, 0)))
    src = edge_index[0].astype(jnp.int32)
    dst = edge_index[1].astype(jnp.int32)
    s0 = src * 2
    s1 = src * 2 + 1
    z = jnp.zeros((NP // NS, H), jnp.float32)

    degp = _deg_call(NP, E)(dst)
    xw1 = _mm1(NP, Cin, C)(xp, W1)
    dis, y1 = _e1(NP, C)(degp, xw1)
    a1lo, a1hi = _agg_call(NP, E, H)(y1.reshape(2 * NP, H), s0, s1, dst, z)
    xw2, y2 = _k2(NP, C, C2)(dis, a1lo, a1hi, xw1, b1.reshape(1, -1), W2)
    a2lo, a2hi = _agg_call(NP, E, C2 // 2)(y2.reshape(2 * NP, C2 // 2),
                                           s0, s1, dst, z)
    out = _k3(NP, C2, O)(dis, a2lo, a2hi, xw2, b2.reshape(1, -1),
                         Wlin, blin.reshape(1, -1))
    return out[:N]
